# baseline (speedup 1.0000x reference)
import jax
import jax.numpy as jnp
from jax.experimental import pallas as pl
from jax.experimental.pallas import tpu as pltpu


NUM_ROWS = 100000
EMBED = 64
BLOCK_ROWS = 5000


def _copy_kernel(w_ref, o_ref):
    o_ref[...] = w_ref[...]


def kernel(x, W):
    del x
    return pl.pallas_call(
        _copy_kernel,
        grid=(NUM_ROWS // BLOCK_ROWS,),
        in_specs=[pl.BlockSpec((BLOCK_ROWS, EMBED), lambda i: (i, 0))],
        out_specs=pl.BlockSpec((BLOCK_ROWS, EMBED), lambda i: (i, 0)),
        out_shape=jax.ShapeDtypeStruct((NUM_ROWS, EMBED), jnp.float32),
        compiler_params=pltpu.CompilerParams(
            dimension_semantics=("parallel",),
        ),
    )(W)
